# baseline (speedup 1.0000x reference)
import functools

import jax
import jax.numpy as jnp
from jax import lax
from jax.experimental import pallas as pl
from jax.experimental.pallas import tpu as pltpu
from jax.experimental.pallas import tpu_sc as plsc

H, W, C = 1024, 1024, 2

NC = 2
NS = 16
L = 16
NW = NC * NS

B = 2048
NSTR = 4 * B // 128
NG = B // L


def _sc_body(npoints, nblocks, coords_hbm, grid_hbm, out_hbm,
             cbuf, ibuf, wxbuf, wybuf, gbuf, obuf, gsem):
    per_worker = npoints // NW
    ids = lax.iota(jnp.int32, L)
    wid = lax.axis_index("s") * NC + lax.axis_index("c")
    base_pt = wid * per_worker

    def block(b, carry):
        blk0 = base_pt + b * B
        pltpu.sync_copy(coords_hbm.at[pl.ds(blk0 * 2, B * 2)], cbuf)

        def p1(gi, _):
            xi = ids * 2 + gi * (2 * L)
            x = plsc.load_gather(cbuf, [xi])
            y = plsc.load_gather(cbuf, [xi + 1])
            xs = x * jnp.float32(W - 1)
            ys = y * jnp.float32(H - 1)
            x0 = xs.astype(jnp.int32)
            y0 = ys.astype(jnp.int32)
            wx = xs - x0.astype(jnp.float32)
            wy = ys - y0.astype(jnp.float32)
            r = y0 * W + x0
            o = gi * L
            ibuf[pl.ds(o, L)] = r
            ibuf[pl.ds(o + B, L)] = r + 1
            ibuf[pl.ds(o + 2 * B, L)] = r + W
            ibuf[pl.ds(o + 3 * B, L)] = r + W + 1
            wxbuf[pl.ds(o, L)] = wx
            wybuf[pl.ds(o, L)] = wy
            return _

        lax.fori_loop(0, NG, p1, 0)

        copies = []
        for j in range(NSTR):
            copies.append(pltpu.async_copy(
                grid_hbm.at[ibuf.at[pl.ds(j * 128, 128)]],
                gbuf.at[pl.ds(j * 128, 128)], gsem))
        for cp in copies:
            cp.wait()

        def p3(gi, _):
            bse = gi * L
            rows = ids + bse
            c0 = jnp.full((L,), 0, jnp.int32)
            c1 = jnp.full((L,), 1, jnp.int32)
            g00_0 = plsc.load_gather(gbuf, [rows, c0])
            g00_1 = plsc.load_gather(gbuf, [rows, c1])
            g01_0 = plsc.load_gather(gbuf, [rows + B, c0])
            g01_1 = plsc.load_gather(gbuf, [rows + B, c1])
            g10_0 = plsc.load_gather(gbuf, [rows + 2 * B, c0])
            g10_1 = plsc.load_gather(gbuf, [rows + 2 * B, c1])
            g11_0 = plsc.load_gather(gbuf, [rows + 3 * B, c0])
            g11_1 = plsc.load_gather(gbuf, [rows + 3 * B, c1])
            wx = wxbuf[pl.ds(bse, L)]
            wy = wybuf[pl.ds(bse, L)]
            top0 = g00_0 + wx * (g01_0 - g00_0)
            top1 = g00_1 + wx * (g01_1 - g00_1)
            bot0 = g10_0 + wx * (g11_0 - g10_0)
            bot1 = g10_1 + wx * (g11_1 - g10_1)
            o0 = top0 + wy * (bot0 - top0)
            o1 = top1 + wy * (bot1 - top1)
            oi = ids * 2 + bse * 2
            plsc.store_scatter(obuf, [oi], o0)
            plsc.store_scatter(obuf, [oi + 1], o1)
            return _

        lax.fori_loop(0, NG, p3, 0)
        pltpu.sync_copy(obuf, out_hbm.at[pl.ds(blk0 * 2, B * 2)])
        return carry

    lax.fori_loop(0, nblocks, block, 0)


def _sample(flat_coords, grid_rows, npoints):
    per_worker = npoints // NW
    nblocks = per_worker // B
    mesh = plsc.VectorSubcoreMesh(core_axis_name="c", subcore_axis_name="s")
    body = functools.partial(_sc_body, npoints, nblocks)
    return pl.kernel(
        body,
        out_type=jax.ShapeDtypeStruct((npoints * 2,), jnp.float32),
        mesh=mesh,
        compiler_params=pltpu.CompilerParams(
            needs_layout_passes=False, use_tc_tiling_on_sc=False),
        scratch_types=[
            pltpu.VMEM((B * 2,), jnp.float32),
            pltpu.VMEM((4 * B,), jnp.int32),
            pltpu.VMEM((B,), jnp.float32),
            pltpu.VMEM((B,), jnp.float32),
            pltpu.VMEM((4 * B, 2), jnp.float32),
            pltpu.VMEM((B * 2,), jnp.float32),
            pltpu.SemaphoreType.DMA,
        ],
    )(flat_coords, grid_rows)


def kernel(coords, vector_field):
    shape = coords.shape
    npoints = coords.size // shape[-1]
    flat_coords = coords.reshape(-1)
    grid_rows = vector_field.reshape(H * W, C)
    out_flat = _sample(flat_coords, grid_rows, npoints)
    return out_flat.reshape(*shape[:-1], C)


# speedup vs baseline: 1.0848x; 1.0848x over previous
import functools

import jax
import jax.numpy as jnp
from jax import lax
from jax.experimental import pallas as pl
from jax.experimental.pallas import tpu as pltpu
from jax.experimental.pallas import tpu_sc as plsc

H, W, C = 1024, 1024, 2

NC = 2
NS = 16
L = 16
NW = NC * NS

RPW = H // NW
RCH = 8
NCH = RPW // RCH

B = 2048
NSTR = B // 128
NG = B // L


def _build_body(grid_hbm, grid8_hbm, rbuf, obuf):
    ids = lax.iota(jnp.int32, L)
    wid = lax.axis_index("s") * NC + lax.axis_index("c")
    row0 = wid * RPW

    def chunk(ck, carry):
        crow = row0 + ck * RCH
        start = jnp.minimum(crow, H - (RCH + 1))
        off = (crow - start) * (2 * W)
        pltpu.sync_copy(grid_hbm.at[pl.ds(start * 2 * W, (RCH + 1) * 2 * W)],
                        rbuf.at[pl.ds(0, (RCH + 1) * 2 * W)])

        def group(g, _):
            ri = g // (W // L)
            gx = g - ri * (W // L)
            x2 = (gx * L + ids) * 2 + ri * (2 * W) + off
            s00_0 = plsc.load_gather(rbuf, [x2])
            s00_1 = plsc.load_gather(rbuf, [x2 + 1])
            s01_0 = plsc.load_gather(rbuf, [x2 + 2])
            s01_1 = plsc.load_gather(rbuf, [x2 + 3])
            s10_0 = plsc.load_gather(rbuf, [x2 + 2 * W])
            s10_1 = plsc.load_gather(rbuf, [x2 + 2 * W + 1])
            s11_0 = plsc.load_gather(rbuf, [x2 + 2 * W + 2])
            s11_1 = plsc.load_gather(rbuf, [x2 + 2 * W + 3])
            o8 = (gx * L + ids) * 8 + ri * (8 * W)
            plsc.store_scatter(obuf, [o8], s00_0)
            plsc.store_scatter(obuf, [o8 + 1], s00_1)
            plsc.store_scatter(obuf, [o8 + 2], s01_0)
            plsc.store_scatter(obuf, [o8 + 3], s01_1)
            plsc.store_scatter(obuf, [o8 + 4], s10_0)
            plsc.store_scatter(obuf, [o8 + 5], s10_1)
            plsc.store_scatter(obuf, [o8 + 6], s11_0)
            plsc.store_scatter(obuf, [o8 + 7], s11_1)
            return _

        lax.fori_loop(0, RCH * (W // L), group, 0)
        pltpu.sync_copy(obuf, grid8_hbm.at[pl.ds(crow * 8 * W, RCH * 8 * W)])
        return carry

    lax.fori_loop(0, NCH, chunk, 0)


def _build_grid8(grid_flat):
    mesh = plsc.VectorSubcoreMesh(core_axis_name="c", subcore_axis_name="s")
    return pl.kernel(
        _build_body,
        out_type=jax.ShapeDtypeStruct((H * W * 8,), jnp.float32),
        mesh=mesh,
        compiler_params=pltpu.CompilerParams(
            needs_layout_passes=False, use_tc_tiling_on_sc=False),
        scratch_types=[
            pltpu.VMEM(((RCH + 2) * 2 * W + 64,), jnp.float32),
            pltpu.VMEM((RCH * 8 * W,), jnp.float32),
        ],
    )(grid_flat)


def _sc_body(npoints, nblocks, coords_hbm, grid8_hbm, out_hbm,
             cbuf, ibuf, wxbuf, wybuf, gbuf, obuf, gsem):
    per_worker = npoints // NW
    ids = lax.iota(jnp.int32, L)
    wid = lax.axis_index("s") * NC + lax.axis_index("c")
    base_pt = wid * per_worker

    def block(b, carry):
        blk0 = base_pt + b * B
        pltpu.sync_copy(coords_hbm.at[pl.ds(blk0 * 2, B * 2)], cbuf)

        def p1(gi, _):
            xi = ids * 2 + gi * (2 * L)
            x = plsc.load_gather(cbuf, [xi])
            y = plsc.load_gather(cbuf, [xi + 1])
            xs = x * jnp.float32(W - 1)
            ys = y * jnp.float32(H - 1)
            x0 = xs.astype(jnp.int32)
            y0 = ys.astype(jnp.int32)
            wx = xs - x0.astype(jnp.float32)
            wy = ys - y0.astype(jnp.float32)
            r = y0 * W + x0
            ibuf[pl.ds(gi * L, L)] = r
            wxbuf[pl.ds(gi * L, L)] = wx
            wybuf[pl.ds(gi * L, L)] = wy
            return _

        lax.fori_loop(0, NG, p1, 0)

        copies = []
        for j in range(NSTR):
            copies.append(pltpu.async_copy(
                grid8_hbm.at[ibuf.at[pl.ds(j * 128, 128)]],
                gbuf.at[pl.ds(j * 128, 128)], gsem))
        for cp in copies:
            cp.wait()

        def p3(gi, _):
            bse = gi * L
            rows = ids + bse
            gv = [plsc.load_gather(gbuf, [rows, jnp.full((L,), k, jnp.int32)])
                  for k in range(8)]
            wx = wxbuf[pl.ds(bse, L)]
            wy = wybuf[pl.ds(bse, L)]
            top0 = gv[0] + wx * (gv[2] - gv[0])
            top1 = gv[1] + wx * (gv[3] - gv[1])
            bot0 = gv[4] + wx * (gv[6] - gv[4])
            bot1 = gv[5] + wx * (gv[7] - gv[5])
            o0 = top0 + wy * (bot0 - top0)
            o1 = top1 + wy * (bot1 - top1)
            oi = ids * 2 + bse * 2
            plsc.store_scatter(obuf, [oi], o0)
            plsc.store_scatter(obuf, [oi + 1], o1)
            return _

        lax.fori_loop(0, NG, p3, 0)
        pltpu.sync_copy(obuf, out_hbm.at[pl.ds(blk0 * 2, B * 2)])
        return carry

    lax.fori_loop(0, nblocks, block, 0)


def _sample(flat_coords, grid8, npoints):
    per_worker = npoints // NW
    nblocks = per_worker // B
    mesh = plsc.VectorSubcoreMesh(core_axis_name="c", subcore_axis_name="s")
    body = functools.partial(_sc_body, npoints, nblocks)
    return pl.kernel(
        body,
        out_type=jax.ShapeDtypeStruct((npoints * 2,), jnp.float32),
        mesh=mesh,
        compiler_params=pltpu.CompilerParams(
            needs_layout_passes=False, use_tc_tiling_on_sc=False),
        scratch_types=[
            pltpu.VMEM((B * 2,), jnp.float32),
            pltpu.VMEM((B,), jnp.int32),
            pltpu.VMEM((B,), jnp.float32),
            pltpu.VMEM((B,), jnp.float32),
            pltpu.VMEM((B, 8), jnp.float32),
            pltpu.VMEM((B * 2,), jnp.float32),
            pltpu.SemaphoreType.DMA,
        ],
    )(flat_coords, grid8)


def kernel(coords, vector_field):
    shape = coords.shape
    npoints = coords.size // shape[-1]
    flat_coords = coords.reshape(-1)
    grid8 = _build_grid8(vector_field.reshape(-1)).reshape(H * W, 8)
    out_flat = _sample(flat_coords, grid8, npoints)
    return out_flat.reshape(*shape[:-1], C)


# speedup vs baseline: 17.2310x; 15.8835x over previous
import functools

import jax
import jax.numpy as jnp
from jax import lax
from jax.experimental import pallas as pl
from jax.experimental.pallas import tpu as pltpu
from jax.experimental.pallas import tpu_sc as plsc

H, W, C = 1024, 1024, 2
T, N1 = 200, 16384

NC = 2
NS = 16
L = 16
NW = NC * NS

RPW = H // NW
RCH = 8
NCH = RPW // RCH

B = 2048
NCK = T * (N1 // B)
CPW = NCK // NW
NSTR = B // 128
NG = B // L
TPL = 2 * N1


def _build_body(grid_hbm, grid8_hbm, rbuf, obuf):
    ids = lax.iota(jnp.int32, L)
    wid = lax.axis_index("s") * NC + lax.axis_index("c")
    row0 = wid * RPW

    def chunk(ck, carry):
        crow = row0 + ck * RCH
        start = jnp.minimum(crow, H - (RCH + 1))
        off = crow - start
        pltpu.sync_copy(grid_hbm.at[pl.ds(start * 2 * W, (RCH + 1) * 2 * W)],
                        rbuf.at[pl.ds(0, (RCH + 1) * 2 * W)])

        def group(g, _):
            ri = g // (W // L)
            gx = g - ri * (W // L)
            x = gx * L + ids
            ro = (ri + off) * (2 * W)
            s00_0 = plsc.load_gather(rbuf, [ro + x])
            s00_1 = plsc.load_gather(rbuf, [ro + W + x])
            s01_0 = plsc.load_gather(rbuf, [ro + x + 1])
            s01_1 = plsc.load_gather(rbuf, [ro + W + x + 1])
            s10_0 = plsc.load_gather(rbuf, [ro + 2 * W + x])
            s10_1 = plsc.load_gather(rbuf, [ro + 3 * W + x])
            s11_0 = plsc.load_gather(rbuf, [ro + 2 * W + x + 1])
            s11_1 = plsc.load_gather(rbuf, [ro + 3 * W + x + 1])
            o8 = x * 8 + ri * (8 * W)
            plsc.store_scatter(obuf, [o8], s00_0)
            plsc.store_scatter(obuf, [o8 + 1], s00_1)
            plsc.store_scatter(obuf, [o8 + 2], s01_0)
            plsc.store_scatter(obuf, [o8 + 3], s01_1)
            plsc.store_scatter(obuf, [o8 + 4], s10_0)
            plsc.store_scatter(obuf, [o8 + 5], s10_1)
            plsc.store_scatter(obuf, [o8 + 6], s11_0)
            plsc.store_scatter(obuf, [o8 + 7], s11_1)
            return _

        lax.fori_loop(0, RCH * (W // L), group, 0)
        pltpu.sync_copy(obuf, grid8_hbm.at[pl.ds(crow * 8 * W, RCH * 8 * W)])
        return carry

    lax.fori_loop(0, NCH, chunk, 0)


def _build_grid8(grid_planar):
    mesh = plsc.VectorSubcoreMesh(core_axis_name="c", subcore_axis_name="s")
    return pl.kernel(
        _build_body,
        out_type=jax.ShapeDtypeStruct((H * W * 8,), jnp.float32),
        mesh=mesh,
        compiler_params=pltpu.CompilerParams(
            needs_layout_passes=False, use_tc_tiling_on_sc=False),
        scratch_types=[
            pltpu.VMEM(((RCH + 2) * 2 * W + 64,), jnp.float32),
            pltpu.VMEM((RCH * 8 * W,), jnp.float32),
        ],
    )(grid_planar)


def _sc_body(coords_hbm, grid8_hbm, out_hbm,
             xbuf, ybuf, ibuf, wxbuf, wybuf, gbuf, oxbuf, oybuf, gsem):
    ids = lax.iota(jnp.int32, L)
    wid = lax.axis_index("s") * NC + lax.axis_index("c")

    def chunk(j, carry):
        u = wid * CPW + j
        t = u // (N1 // B)
        base = t * TPL + (u - t * (N1 // B)) * B
        pltpu.sync_copy(coords_hbm.at[pl.ds(base, B)], xbuf)
        pltpu.sync_copy(coords_hbm.at[pl.ds(base + N1, B)], ybuf)

        def p1(gi, _):
            x = xbuf[pl.ds(gi * L, L)]
            y = ybuf[pl.ds(gi * L, L)]
            xs = x * jnp.float32(W - 1)
            ys = y * jnp.float32(H - 1)
            x0 = xs.astype(jnp.int32)
            y0 = ys.astype(jnp.int32)
            wx = xs - x0.astype(jnp.float32)
            wy = ys - y0.astype(jnp.float32)
            ibuf[pl.ds(gi * L, L)] = y0 * W + x0
            wxbuf[pl.ds(gi * L, L)] = wx
            wybuf[pl.ds(gi * L, L)] = wy
            return _

        lax.fori_loop(0, NG, p1, 0)

        copies = []
        for s in range(NSTR):
            copies.append(pltpu.async_copy(
                grid8_hbm.at[ibuf.at[pl.ds(s * 128, 128)]],
                gbuf.at[pl.ds(s * 128, 128)], gsem))
        for cp in copies:
            cp.wait()

        def p3(gi, _):
            bse = gi * L
            rows = ids + bse
            gv = [plsc.load_gather(gbuf, [rows, jnp.full((L,), k, jnp.int32)])
                  for k in range(8)]
            wx = wxbuf[pl.ds(bse, L)]
            wy = wybuf[pl.ds(bse, L)]
            top0 = gv[0] + wx * (gv[2] - gv[0])
            top1 = gv[1] + wx * (gv[3] - gv[1])
            bot0 = gv[4] + wx * (gv[6] - gv[4])
            bot1 = gv[5] + wx * (gv[7] - gv[5])
            oxbuf[pl.ds(bse, L)] = top0 + wy * (bot0 - top0)
            oybuf[pl.ds(bse, L)] = top1 + wy * (bot1 - top1)
            return _

        lax.fori_loop(0, NG, p3, 0)
        pltpu.sync_copy(oxbuf, out_hbm.at[pl.ds(base, B)])
        pltpu.sync_copy(oybuf, out_hbm.at[pl.ds(base + N1, B)])
        return carry

    lax.fori_loop(0, CPW, chunk, 0)


def _sample(coords_planar, grid8):
    mesh = plsc.VectorSubcoreMesh(core_axis_name="c", subcore_axis_name="s")
    return pl.kernel(
        _sc_body,
        out_type=jax.ShapeDtypeStruct((T * 2 * N1,), jnp.float32),
        mesh=mesh,
        compiler_params=pltpu.CompilerParams(
            needs_layout_passes=False, use_tc_tiling_on_sc=False),
        scratch_types=[
            pltpu.VMEM((B,), jnp.float32),
            pltpu.VMEM((B,), jnp.float32),
            pltpu.VMEM((B,), jnp.int32),
            pltpu.VMEM((B,), jnp.float32),
            pltpu.VMEM((B,), jnp.float32),
            pltpu.VMEM((B, 8), jnp.float32),
            pltpu.VMEM((B,), jnp.float32),
            pltpu.VMEM((B,), jnp.float32),
            pltpu.SemaphoreType.DMA,
        ],
    )(coords_planar, grid8)


def kernel(coords, vector_field):
    coords_planar = jnp.transpose(coords, (1, 2, 0)).reshape(-1)
    grid_planar = jnp.transpose(vector_field, (0, 2, 1)).reshape(-1)
    grid8 = _build_grid8(grid_planar).reshape(H * W, 8)
    out_flat = _sample(coords_planar, grid8)
    return jnp.transpose(out_flat.reshape(T, 2, N1), (2, 0, 1))


# speedup vs baseline: 23.5700x; 1.3679x over previous
import functools

import jax
import jax.numpy as jnp
from jax import lax
from jax.experimental import pallas as pl
from jax.experimental.pallas import tpu as pltpu
from jax.experimental.pallas import tpu_sc as plsc

H, W, C = 1024, 1024, 2
T, N1 = 200, 16384

NC = 2
NS = 16
L = 16
NW = NC * NS

RPW = H // NW
RCH = 8
NCH = RPW // RCH

B = 2048
NCK = T * (N1 // B)
CPW = NCK // NW
NSTR = B // 128
NG = B // L
TPL = 2 * N1


def _build_body(grid_hbm, grid8_hbm, rbuf, obuf):
    ids = lax.iota(jnp.int32, L)
    wid = lax.axis_index("s") * NC + lax.axis_index("c")
    row0 = wid * RPW

    def chunk(ck, carry):
        crow = row0 + ck * RCH
        start = jnp.minimum(crow, H - (RCH + 1))
        off = crow - start
        pltpu.sync_copy(grid_hbm.at[pl.ds(start * 2 * W, (RCH + 1) * 2 * W)],
                        rbuf.at[pl.ds(0, (RCH + 1) * 2 * W)])

        def group(g, _):
            ri = g // (W // L)
            gx = g - ri * (W // L)
            x = gx * L + ids
            ro = (ri + off) * (2 * W)
            s00_0 = plsc.load_gather(rbuf, [ro + x])
            s00_1 = plsc.load_gather(rbuf, [ro + W + x])
            s01_0 = plsc.load_gather(rbuf, [ro + x + 1])
            s01_1 = plsc.load_gather(rbuf, [ro + W + x + 1])
            s10_0 = plsc.load_gather(rbuf, [ro + 2 * W + x])
            s10_1 = plsc.load_gather(rbuf, [ro + 3 * W + x])
            s11_0 = plsc.load_gather(rbuf, [ro + 2 * W + x + 1])
            s11_1 = plsc.load_gather(rbuf, [ro + 3 * W + x + 1])
            o8 = x * 8 + ri * (8 * W)
            plsc.store_scatter(obuf, [o8], s00_0)
            plsc.store_scatter(obuf, [o8 + 1], s00_1)
            plsc.store_scatter(obuf, [o8 + 2], s01_0)
            plsc.store_scatter(obuf, [o8 + 3], s01_1)
            plsc.store_scatter(obuf, [o8 + 4], s10_0)
            plsc.store_scatter(obuf, [o8 + 5], s10_1)
            plsc.store_scatter(obuf, [o8 + 6], s11_0)
            plsc.store_scatter(obuf, [o8 + 7], s11_1)
            return _

        lax.fori_loop(0, RCH * (W // L), group, 0)
        pltpu.sync_copy(obuf, grid8_hbm.at[pl.ds(crow * 8 * W, RCH * 8 * W)])
        return carry

    lax.fori_loop(0, NCH, chunk, 0)


def _build_grid8(grid_planar):
    mesh = plsc.VectorSubcoreMesh(core_axis_name="c", subcore_axis_name="s")
    return pl.kernel(
        _build_body,
        out_type=jax.ShapeDtypeStruct((H * W * 8,), jnp.float32),
        mesh=mesh,
        compiler_params=pltpu.CompilerParams(
            needs_layout_passes=False, use_tc_tiling_on_sc=False),
        scratch_types=[
            pltpu.VMEM(((RCH + 2) * 2 * W + 64,), jnp.float32),
            pltpu.VMEM((RCH * 8 * W,), jnp.float32),
        ],
    )(grid_planar)


NPT = N1 // B


def _sc_body(coords_hbm, grid8_hbm, out_hbm,
             xb0, xb1, yb0, yb1, ib0, ib1, wx0, wx1, wy0, wy1,
             gb0, gb1, ox0, ox1, oy0, oy1, csem, gsem, osem):
    ids = lax.iota(jnp.int32, L)
    wid = lax.axis_index("s") * NC + lax.axis_index("c")
    u0 = wid * CPW
    xb, yb, ib = (xb0, xb1), (yb0, yb1), (ib0, ib1)
    wxb, wyb = (wx0, wx1), (wy0, wy1)
    gb, oxb, oyb = (gb0, gb1), (ox0, ox1), (oy0, oy1)

    def addr(u):
        t = u // NPT
        return t * TPL + (u - t * NPT) * B

    def p1(xbuf, ybuf, ibuf, wxbuf, wybuf):
        def f(gi, carry):
            x = xbuf[pl.ds(gi * L, L)]
            y = ybuf[pl.ds(gi * L, L)]
            xs = x * jnp.float32(W - 1)
            ys = y * jnp.float32(H - 1)
            x0 = xs.astype(jnp.int32)
            y0 = ys.astype(jnp.int32)
            ibuf[pl.ds(gi * L, L)] = y0 * W + x0
            wxbuf[pl.ds(gi * L, L)] = xs - x0.astype(jnp.float32)
            wybuf[pl.ds(gi * L, L)] = ys - y0.astype(jnp.float32)
            return carry

        lax.fori_loop(0, NG, f, 0, unroll=4)

    def p3(gbuf, wxbuf, wybuf, oxbuf, oybuf):
        def f(gi, carry):
            bse = gi * L
            rows = ids + bse
            gv = [plsc.load_gather(gbuf, [rows, jnp.full((L,), k, jnp.int32)])
                  for k in range(8)]
            wx = wxbuf[pl.ds(bse, L)]
            wy = wybuf[pl.ds(bse, L)]
            top0 = gv[0] + wx * (gv[2] - gv[0])
            top1 = gv[1] + wx * (gv[3] - gv[1])
            bot0 = gv[4] + wx * (gv[6] - gv[4])
            bot1 = gv[5] + wx * (gv[7] - gv[5])
            oxbuf[pl.ds(bse, L)] = top0 + wy * (bot0 - top0)
            oybuf[pl.ds(bse, L)] = top1 + wy * (bot1 - top1)
            return carry

        lax.fori_loop(0, NG, f, 0, unroll=4)

    b0 = addr(u0)
    pltpu.async_copy(coords_hbm.at[pl.ds(b0, B)], xb[0], csem)
    pltpu.async_copy(coords_hbm.at[pl.ds(b0 + N1, B)], yb[0], csem)

    def iteration(k, cur):
        nxt = 1 - cur
        base = addr(u0 + k)

        @pl.when(k + 1 < CPW)
        def _():
            bn = addr(u0 + k + 1)
            pltpu.async_copy(coords_hbm.at[pl.ds(bn, B)], xb[nxt], csem)
            pltpu.async_copy(coords_hbm.at[pl.ds(bn + N1, B)], yb[nxt], csem)

        pltpu.make_async_copy(coords_hbm.at[pl.ds(base, B)], xb[cur], csem).wait()
        pltpu.make_async_copy(coords_hbm.at[pl.ds(base + N1, B)], yb[cur], csem).wait()

        p1(xb[cur], yb[cur], ib[cur], wxb[cur], wyb[cur])
        for s in range(NSTR):
            pltpu.async_copy(
                grid8_hbm.at[ib[cur].at[pl.ds(s * 128, 128)]],
                gb[cur].at[pl.ds(s * 128, 128)], gsem)

        @pl.when(k >= 1)
        def _():
            @pl.when(k >= 2)
            def _():
                bq = addr(u0 + k - 2)
                pltpu.make_async_copy(oxb[cur], out_hbm.at[pl.ds(bq, B)], osem).wait()
                pltpu.make_async_copy(oyb[cur], out_hbm.at[pl.ds(bq + N1, B)], osem).wait()
            p3(gb[nxt], wxb[nxt], wyb[nxt], oxb[nxt], oyb[nxt])
            bp = addr(u0 + k - 1)
            pltpu.async_copy(oxb[nxt], out_hbm.at[pl.ds(bp, B)], osem)
            pltpu.async_copy(oyb[nxt], out_hbm.at[pl.ds(bp + N1, B)], osem)

        for s in range(NSTR):
            pltpu.make_async_copy(
                grid8_hbm.at[ib[cur].at[pl.ds(s * 128, 128)]],
                gb[cur].at[pl.ds(s * 128, 128)], gsem).wait()

    def two(j2, carry):
        iteration(j2 * 2, 0)
        iteration(j2 * 2 + 1, 1)
        return carry

    lax.fori_loop(0, CPW // 2, two, 0)

    lastp = (CPW - 1) % 2
    p3(gb[lastp], wxb[lastp], wyb[lastp], oxb[lastp], oyb[lastp])
    bl = addr(u0 + CPW - 1)
    pltpu.async_copy(oxb[lastp], out_hbm.at[pl.ds(bl, B)], osem)
    pltpu.async_copy(oyb[lastp], out_hbm.at[pl.ds(bl + N1, B)], osem)
    for _i in range(4):
        pltpu.make_async_copy(oxb[0], out_hbm.at[pl.ds(bl, B)], osem).wait()


def _sample(coords_planar, grid8):
    mesh = plsc.VectorSubcoreMesh(core_axis_name="c", subcore_axis_name="s")
    return pl.kernel(
        _sc_body,
        out_type=jax.ShapeDtypeStruct((T * 2 * N1,), jnp.float32),
        mesh=mesh,
        compiler_params=pltpu.CompilerParams(
            needs_layout_passes=False, use_tc_tiling_on_sc=False),
        scratch_types=(
            [pltpu.VMEM((B,), jnp.float32)] * 4
            + [pltpu.VMEM((B,), jnp.int32)] * 2
            + [pltpu.VMEM((B,), jnp.float32)] * 4
            + [pltpu.VMEM((B, 8), jnp.float32)] * 2
            + [pltpu.VMEM((B,), jnp.float32)] * 4
            + [pltpu.SemaphoreType.DMA] * 3
        ),
    )(coords_planar, grid8)


def kernel(coords, vector_field):
    coords_planar = jnp.transpose(coords, (1, 2, 0)).reshape(-1)
    grid_planar = jnp.transpose(vector_field, (0, 2, 1)).reshape(-1)
    grid8 = _build_grid8(grid_planar).reshape(H * W, 8)
    out_flat = _sample(coords_planar, grid8)
    return jnp.transpose(out_flat.reshape(T, 2, N1), (2, 0, 1))
